# Initial kernel scaffold; baseline (speedup 1.0000x reference)
import jax
import jax.numpy as jnp
from jax.experimental import pallas as pl


def kernel(x, edge_attr, senders, receivers, u, We1, Ws1, Wr1, Wg1, be1, Wn1, Win1, Wgn1, bn1, WGn1, WGe1, WGg1, bg1, We2, Ws2, Wr2, Wg2, be2, Wn2, Win2, Wgn2, bn2, WGn2, WGe2, WGg2, bg2):
    raise NotImplementedError("write your pallas kernel here")


# speedup vs baseline: 1.5959x; 1.5959x over previous
import functools

import jax
import jax.numpy as jnp
from jax import lax
from jax.experimental import pallas as pl
from jax.experimental.pallas import tpu as pltpu
from jax.experimental.pallas import tpu_sc as plsc

N = 10000
E = 320000
DN = 128
DE = 16
DG = 32
H1 = 256
H2 = 128
G1 = 16
HH = 128

NH = N // 2
NHP = NH + 8
NC = 2
NS = 16
SCB = 64
ECH1 = E // SCB
ECPS1 = -(-ECH1 // NS)
E2 = E // NC
ECH2 = E2 // SCB
ECPS2 = -(-ECH2 // NS)
CP = 40
NCH = NH // CP
CPS = -(-NCH // NS)

_DOT = functools.partial(jnp.dot, precision=jax.lax.Precision.HIGHEST)


def _relu(v):
    return jnp.maximum(v, 0.0)


def _proj1_body(x_ref, ws_ref, wr_ref, xs_ref, xr_ref):
    m = _DOT(x_ref[...], ws_ref[...])
    xs_ref[0] = m[:, :HH]
    xs_ref[1] = m[:, HH:]
    m = _DOT(x_ref[...], wr_ref[...])
    xr_ref[0] = m[:, :HH]
    xr_ref[1] = m[:, HH:]


def _zedge_body(ea_ref, we_ref, u_ref, wg_ref, be_ref, z_ref):
    c1 = _DOT(u_ref[...], wg_ref[...]) + be_ref[...]
    z = _DOT(ea_ref[...], we_ref[...]) + c1
    z_ref[0] = z[:, :HH]
    z_ref[1] = z[:, HH:]


def _node1_body(x_ref, seg_ref, cnt_ref, u_ref, wn_ref, win_ref, wgn_ref,
                bn_ref, ws2_ref, wr2_ref,
                n1_ref, ns2_ref, nr2_ref, nsum_ref, esum_ref):
    i = pl.program_id(0)

    @pl.when(i == 0)
    def _():
        nsum_ref[...] = jnp.zeros_like(nsum_ref)
        esum_ref[...] = jnp.zeros_like(esum_ref)

    cb = jnp.maximum(cnt_ref[:, 0:1], 1.0)
    agg_lo = seg_ref[0] / cb
    agg_hi = seg_ref[1] / cb
    pre = (_DOT(x_ref[...], wn_ref[...])
           + _DOT(agg_lo, win_ref[:HH, :])
           + _DOT(agg_hi, win_ref[HH:, :])
           + _DOT(u_ref[...], wgn_ref[...]) + bn_ref[...])
    n1 = _relu(pre)
    n1_ref[...] = n1
    ns2_ref[...] = _DOT(n1, ws2_ref[...])
    nr2_ref[...] = _DOT(n1, wr2_ref[...])
    nsum_ref[...] += jnp.sum(n1, axis=0, keepdims=True)
    esum_ref[...] += jnp.concatenate(
        [jnp.sum(seg_ref[0], axis=0, keepdims=True),
         jnp.sum(seg_ref[1], axis=0, keepdims=True)], axis=1)


def _glob1_body(nsum_ref, esum_ref, u_ref, wgn1_ref, wge1_ref, wgg1_ref,
                bg1_ref, wg2_ref, be2_ref, wgn2_ref, bn2_ref,
                u1_ref, c2_ref, cn2_ref):
    u1 = _relu(_DOT(nsum_ref[...] * (1.0 / N), wgn1_ref[...])
               + _DOT(esum_ref[...] * (1.0 / E), wge1_ref[...])
               + _DOT(u_ref[...], wgg1_ref[...]) + bg1_ref[...])
    u1_ref[...] = u1
    c2_ref[...] = _DOT(u1, wg2_ref[...]) + be2_ref[...]
    cn2_ref[...] = _DOT(u1, wgn2_ref[...]) + bn2_ref[...]


def _t2_body(e1_ref, we2_ref, c2_ref, t2_ref):
    t2_ref[...] = (_DOT(e1_ref[0], we2_ref[:HH, :])
                   + _DOT(e1_ref[1], we2_ref[HH:, :]) + c2_ref[...])


def _node2_body(n1_ref, seg_ref, cnt_ref, wn2_ref, win2_ref, cn2_ref,
                n2sum_ref, esum2_ref):
    i = pl.program_id(0)

    @pl.when(i == 0)
    def _():
        n2sum_ref[...] = jnp.zeros_like(n2sum_ref)
        esum2_ref[...] = jnp.zeros_like(esum2_ref)

    cb = jnp.maximum(cnt_ref[:, 0:1], 1.0)
    sseg = seg_ref[0] + seg_ref[1]
    agg2 = sseg / cb
    n2 = _relu(_DOT(n1_ref[...], wn2_ref[...])
               + _DOT(agg2, win2_ref[...]) + cn2_ref[...])
    n2sum_ref[...] += jnp.sum(n2, axis=0, keepdims=True)
    esum2_ref[...] += jnp.sum(sseg, axis=0, keepdims=True)


def _out_body(n2sum_ref, esum2_ref, u1_ref, wgn2_ref, wge2_ref, wgg2_ref,
              bg2_ref, o_ref):
    o_ref[...] = (_DOT(n2sum_ref[...] * (1.0 / N), wgn2_ref[...])
                  + _DOT(esum2_ref[...] * (1.0 / E), wge2_ref[...])
                  + _DOT(u1_ref[...], wgg2_ref[...]) + bg2_ref[...])


def _full(shape):
    return pl.BlockSpec(shape, lambda *_: (0,) * len(shape))


def _zero_vmem(ref, rows, cols):
    def body(j, carry):
        for k in range(cols // 16):
            ref[j, pl.ds(k * 16, 16)] = jnp.zeros((16,), jnp.float32)
        return carry
    lax.fori_loop(0, rows, body, 0)


def _sc_pass1(z_hbm, xs_hbm, xr_hbm, snd2_hbm, rcv2_hbm, rcvc_hbm,
              e1_hbm, seg_hbm,
              acc_sh,
              sidx, gidx, cidx,
              zb, xsb, xrb,
              sem0, sem1, sem2):
    c = lax.axis_index("c")
    s = lax.axis_index("s")

    def init_acc():
        for kk in range(CPS):
            t = s * CPS + kk

            @pl.when(t < NCH)
            def _():
                pltpu.sync_copy(zb.at[pl.ds(0, CP)],
                                acc_sh.at[pl.ds(t * CP, CP)])

    _zero_vmem(zb, CP, HH)
    init_acc()
    plsc.subcore_barrier()

    for h in range(2):

        def step(i, carry):
            t = s * ECPS1 + i

            @pl.when(t < ECH1)
            def _():
                base = t * SCB
                pltpu.sync_copy(rcvc_hbm.at[pl.ds(h * E + base, SCB)], cidx)
                if h == 0:
                    pltpu.sync_copy(snd2_hbm.at[pl.ds(c * E + base, SCB)],
                                    sidx)
                    pltpu.sync_copy(rcv2_hbm.at[pl.ds(c * E + base, SCB)],
                                    gidx)
                    cp1 = pltpu.async_copy(xs_hbm.at[sidx], xsb, sem0)
                    cp2 = pltpu.async_copy(xr_hbm.at[gidx], xrb, sem1)
                    cp3 = pltpu.async_copy(
                        z_hbm.at[pl.ds(c * E + base, SCB)], zb, sem2)
                    cp1.wait()
                    cp2.wait()
                    cp3.wait()

                    def row(j, carry2):
                        for k in range(HH // 16):
                            sl = pl.ds(k * 16, 16)
                            zb[j, sl] = jnp.maximum(
                                zb[j, sl] + xsb[j, sl] + xrb[j, sl], 0.0)
                        return carry2
                    lax.fori_loop(0, SCB, row, 0)

                    pltpu.sync_copy(zb, e1_hbm.at[pl.ds(c * E + base, SCB)])
                else:
                    pltpu.sync_copy(e1_hbm.at[pl.ds(c * E + base, SCB)], zb)

                pltpu.sync_copy(zb, acc_sh.at[cidx], add=True)
            return carry

        lax.fori_loop(0, ECPS1, step, 0)
        plsc.subcore_barrier()

        for kk in range(CPS):
            t = s * CPS + kk

            @pl.when(t < NCH)
            def _():
                pltpu.sync_copy(
                    acc_sh.at[pl.ds(t * CP, CP)],
                    seg_hbm.at[pl.ds(c * N + h * NH + t * CP, CP)])

        if h == 0:
            _zero_vmem(zb, CP, HH)
            init_acc()
            plsc.subcore_barrier()


def _sc_pass2(t2_hbm, ns2_hbm, nr2_hbm, snd_hbm, rcv_hbm, rcvc_hbm,
              e2_hbm, seg_hbm,
              acc_sh,
              sidx, gidx, cidx,
              tb, nsb, nrb,
              sem0, sem1, sem2):
    c = lax.axis_index("c")
    s = lax.axis_index("s")

    def init_acc():
        for kk in range(CPS):
            t = s * CPS + kk

            @pl.when(t < NCH)
            def _():
                pltpu.sync_copy(tb.at[pl.ds(0, CP)],
                                acc_sh.at[pl.ds(t * CP, CP)])

    _zero_vmem(tb, CP, HH)
    init_acc()
    plsc.subcore_barrier()

    for h in range(2):

        def step(i, carry):
            t = s * ECPS2 + i

            @pl.when(t < ECH2)
            def _():
                base = c * E2 + t * SCB
                pltpu.sync_copy(rcvc_hbm.at[pl.ds(h * E + base, SCB)], cidx)
                if h == 0:
                    pltpu.sync_copy(snd_hbm.at[pl.ds(base, SCB)], sidx)
                    pltpu.sync_copy(rcv_hbm.at[pl.ds(base, SCB)], gidx)
                    cp1 = pltpu.async_copy(ns2_hbm.at[sidx], nsb, sem0)
                    cp2 = pltpu.async_copy(nr2_hbm.at[gidx], nrb, sem1)
                    cp3 = pltpu.async_copy(t2_hbm.at[pl.ds(base, SCB)], tb,
                                           sem2)
                    cp1.wait()
                    cp2.wait()
                    cp3.wait()

                    def row(j, carry2):
                        for k in range(HH // 16):
                            sl = pl.ds(k * 16, 16)
                            tb[j, sl] = jnp.maximum(
                                tb[j, sl] + nsb[j, sl] + nrb[j, sl], 0.0)
                        return carry2
                    lax.fori_loop(0, SCB, row, 0)

                    pltpu.sync_copy(tb, e2_hbm.at[pl.ds(base, SCB)])
                else:
                    pltpu.sync_copy(e2_hbm.at[pl.ds(base, SCB)], tb)

                pltpu.sync_copy(tb, acc_sh.at[cidx], add=True)
            return carry

        lax.fori_loop(0, ECPS2, step, 0)
        plsc.subcore_barrier()

        for kk in range(CPS):
            t = s * CPS + kk

            @pl.when(t < NCH)
            def _():
                pltpu.sync_copy(
                    acc_sh.at[pl.ds(t * CP, CP)],
                    seg_hbm.at[pl.ds(c * N + h * NH + t * CP, CP)])

        if h == 0:
            _zero_vmem(tb, CP, HH)
            init_acc()
            plsc.subcore_barrier()


def _sc_cnt(rcvc_hbm, cnt_hbm, acc_sh, cidx, onesb, sem0):
    c = lax.axis_index("c")
    s = lax.axis_index("s")

    _zero_vmem(onesb, CP, HH)
    for kk in range(CPS):
        t = s * CPS + kk

        @pl.when(t < NCH)
        def _():
            pltpu.sync_copy(onesb.at[pl.ds(0, CP)],
                            acc_sh.at[pl.ds(t * CP, CP)])

    def ones_body(j, carry):
        for k in range(HH // 16):
            onesb[j, pl.ds(k * 16, 16)] = jnp.full((16,), 1.0, jnp.float32)
        return carry
    lax.fori_loop(0, SCB, ones_body, 0)
    plsc.subcore_barrier()

    def step(i, carry):
        t = s * ECPS1 + i

        @pl.when(t < ECH1)
        def _():
            base = t * SCB
            pltpu.sync_copy(rcvc_hbm.at[pl.ds(c * E + base, SCB)], cidx)
            pltpu.sync_copy(onesb, acc_sh.at[cidx], add=True)
        return carry

    lax.fori_loop(0, ECPS1, step, 0)
    plsc.subcore_barrier()

    for kk in range(CPS):
        t = s * CPS + kk

        @pl.when(t < NCH)
        def _():
            pltpu.sync_copy(acc_sh.at[pl.ds(t * CP, CP)],
                            cnt_hbm.at[pl.ds(c * NH + t * CP, CP)])


_sc_calls = {}


def _build_sc_calls():
    if _sc_calls:
        return
    mesh = plsc.VectorSubcoreMesh(core_axis_name="c", subcore_axis_name="s")
    _sc_calls["cnt"] = pl.kernel(
        _sc_cnt,
        out_type=[
            jax.ShapeDtypeStruct((N, HH), jnp.float32),
        ],
        mesh=mesh,
        scratch_types=[
            pltpu.VMEM_SHARED((NHP, HH), jnp.float32),
            pltpu.VMEM((SCB,), jnp.int32),
            pltpu.VMEM((SCB, HH), jnp.float32),
            pltpu.SemaphoreType.DMA,
        ],
    )
    _sc_calls["p1"] = pl.kernel(
        _sc_pass1,
        out_type=[
            jax.ShapeDtypeStruct((2 * E, HH), jnp.float32),
            jax.ShapeDtypeStruct((2 * N, HH), jnp.float32),
        ],
        mesh=mesh,
        scratch_types=[
            pltpu.VMEM_SHARED((NHP, HH), jnp.float32),
            pltpu.VMEM((SCB,), jnp.int32),
            pltpu.VMEM((SCB,), jnp.int32),
            pltpu.VMEM((SCB,), jnp.int32),
            pltpu.VMEM((SCB, HH), jnp.float32),
            pltpu.VMEM((SCB, HH), jnp.float32),
            pltpu.VMEM((SCB, HH), jnp.float32),
            pltpu.SemaphoreType.DMA,
            pltpu.SemaphoreType.DMA,
            pltpu.SemaphoreType.DMA,
        ],
    )
    _sc_calls["p2"] = pl.kernel(
        _sc_pass2,
        out_type=[
            jax.ShapeDtypeStruct((E, HH), jnp.float32),
            jax.ShapeDtypeStruct((2 * N, HH), jnp.float32),
        ],
        mesh=mesh,
        scratch_types=[
            pltpu.VMEM_SHARED((NHP, HH), jnp.float32),
            pltpu.VMEM((SCB,), jnp.int32),
            pltpu.VMEM((SCB,), jnp.int32),
            pltpu.VMEM((SCB,), jnp.int32),
            pltpu.VMEM((SCB, HH), jnp.float32),
            pltpu.VMEM((SCB, HH), jnp.float32),
            pltpu.VMEM((SCB, HH), jnp.float32),
            pltpu.SemaphoreType.DMA,
            pltpu.SemaphoreType.DMA,
            pltpu.SemaphoreType.DMA,
        ],
    )


def kernel(x, edge_attr, senders, receivers, u,
           We1, Ws1, Wr1, Wg1, be1,
           Wn1, Win1, Wgn1, bn1,
           WGn1, WGe1, WGg1, bg1,
           We2, Ws2, Wr2, Wg2, be2,
           Wn2, Win2, Wgn2, bn2,
           WGn2, WGe2, WGg2, bg2):
    _build_sc_calls()
    u2 = u.reshape(1, DG)
    be1r = be1.reshape(1, H1)
    bn1r = bn1.reshape(1, H1)
    bg1r = bg1.reshape(1, G1)
    be2r = be2.reshape(1, H2)
    bn2r = bn2.reshape(1, H2)
    bg2r = bg2.reshape(1, 1)
    snd = senders.astype(jnp.int32)
    rcv = receivers.astype(jnp.int32)
    snd2 = jnp.concatenate([snd, snd + N])
    rcv2 = jnp.concatenate([rcv, rcv + N])
    rcvc = jnp.concatenate([jnp.where(rcv < NH, rcv, NH),
                            jnp.where(rcv >= NH, rcv - NH, NH)])

    BN = 1000
    BEB = 2000

    xs3, xr3 = pl.pallas_call(
        _proj1_body,
        grid=(N // BN,),
        in_specs=[pl.BlockSpec((BN, DN), lambda i: (i, 0)),
                  _full((DN, H1)), _full((DN, H1))],
        out_specs=[pl.BlockSpec((2, BN, HH), lambda i: (0, i, 0)),
                   pl.BlockSpec((2, BN, HH), lambda i: (0, i, 0))],
        out_shape=[jax.ShapeDtypeStruct((2, N, HH), jnp.float32),
                   jax.ShapeDtypeStruct((2, N, HH), jnp.float32)],
    )(x, Ws1, Wr1)

    z3 = pl.pallas_call(
        _zedge_body,
        grid=(E // BEB,),
        in_specs=[pl.BlockSpec((BEB, DE), lambda i: (i, 0)),
                  _full((DE, H1)), _full((1, DG)), _full((DG, H1)),
                  _full((1, H1))],
        out_specs=pl.BlockSpec((2, BEB, HH), lambda i: (0, i, 0)),
        out_shape=jax.ShapeDtypeStruct((2, E, HH), jnp.float32),
    )(edge_attr, We1, u2, Wg1, be1r)

    (cnt,) = _sc_calls["cnt"](rcvc)

    e1f, seg1f = _sc_calls["p1"](
        z3.reshape(2 * E, HH), xs3.reshape(2 * N, HH), xr3.reshape(2 * N, HH),
        snd2, rcv2, rcvc)
    e1_3 = e1f.reshape(2, E, HH)
    seg1_3 = seg1f.reshape(2, N, HH)

    n1, ns2, nr2, nsum, esum1 = pl.pallas_call(
        _node1_body,
        grid=(N // BN,),
        in_specs=[pl.BlockSpec((BN, DN), lambda i: (i, 0)),
                  pl.BlockSpec((2, BN, HH), lambda i: (0, i, 0)),
                  pl.BlockSpec((BN, HH), lambda i: (i, 0)),
                  _full((1, DG)), _full((DN, H1)), _full((H1, H1)),
                  _full((DG, H1)), _full((1, H1)), _full((H1, H2)),
                  _full((H1, H2))],
        out_specs=[pl.BlockSpec((BN, H1), lambda i: (i, 0)),
                   pl.BlockSpec((BN, H2), lambda i: (i, 0)),
                   pl.BlockSpec((BN, H2), lambda i: (i, 0)),
                   pl.BlockSpec((1, H1), lambda i: (0, 0)),
                   pl.BlockSpec((1, H1), lambda i: (0, 0))],
        out_shape=[jax.ShapeDtypeStruct((N, H1), jnp.float32),
                   jax.ShapeDtypeStruct((N, H2), jnp.float32),
                   jax.ShapeDtypeStruct((N, H2), jnp.float32),
                   jax.ShapeDtypeStruct((1, H1), jnp.float32),
                   jax.ShapeDtypeStruct((1, H1), jnp.float32)],
    )(x, seg1_3, cnt, u2, Wn1, Win1, Wgn1, bn1r, Ws2, Wr2)

    u1, c2, cn2 = pl.pallas_call(
        _glob1_body,
        in_specs=[_full((1, H1)), _full((1, H1)), _full((1, DG)),
                  _full((H1, G1)), _full((H1, G1)), _full((DG, G1)),
                  _full((1, G1)), _full((G1, H2)), _full((1, H2)),
                  _full((G1, H2)), _full((1, H2))],
        out_specs=[_full((1, G1)), _full((1, H2)), _full((1, H2))],
        out_shape=[jax.ShapeDtypeStruct((1, G1), jnp.float32),
                   jax.ShapeDtypeStruct((1, H2), jnp.float32),
                   jax.ShapeDtypeStruct((1, H2), jnp.float32)],
    )(nsum, esum1, u2, WGn1, WGe1, WGg1, bg1r, Wg2, be2r, Wgn2, bn2r)

    t2 = pl.pallas_call(
        _t2_body,
        grid=(E // BEB,),
        in_specs=[pl.BlockSpec((2, BEB, HH), lambda i: (0, i, 0)),
                  _full((H1, H2)), _full((1, H2))],
        out_specs=pl.BlockSpec((BEB, H2), lambda i: (i, 0)),
        out_shape=jax.ShapeDtypeStruct((E, H2), jnp.float32),
    )(e1_3, We2, c2)

    _, seg2f = _sc_calls["p2"](t2, ns2, nr2, snd, rcv, rcvc)
    seg2_2 = seg2f.reshape(2, N, HH)

    n2sum, esum2 = pl.pallas_call(
        _node2_body,
        grid=(N // BN,),
        in_specs=[pl.BlockSpec((BN, H1), lambda i: (i, 0)),
                  pl.BlockSpec((2, BN, HH), lambda i: (0, i, 0)),
                  pl.BlockSpec((BN, HH), lambda i: (i, 0)),
                  _full((H1, H2)), _full((H2, H2)), _full((1, H2))],
        out_specs=[pl.BlockSpec((1, H2), lambda i: (0, 0)),
                   pl.BlockSpec((1, H2), lambda i: (0, 0))],
        out_shape=[jax.ShapeDtypeStruct((1, H2), jnp.float32),
                   jax.ShapeDtypeStruct((1, H2), jnp.float32)],
    )(n1, seg2_2, cnt, Wn2, Win2, cn2)

    o = pl.pallas_call(
        _out_body,
        in_specs=[_full((1, H2)), _full((1, H2)), _full((1, G1)),
                  _full((H2, 1)), _full((H2, 1)), _full((G1, 1)),
                  _full((1, 1))],
        out_specs=_full((1, 1)),
        out_shape=jax.ShapeDtypeStruct((1, 1), jnp.float32),
    )(n2sum, esum2, u1, WGn2, WGe2, WGg2, bg2r)

    return o.reshape(1)


# speedup vs baseline: 1.7113x; 1.0723x over previous
import functools

import jax
import jax.numpy as jnp
from jax import lax
from jax.experimental import pallas as pl
from jax.experimental.pallas import tpu as pltpu
from jax.experimental.pallas import tpu_sc as plsc

N = 10000
E = 320000
DN = 128
DE = 16
DG = 32
H1 = 256
H2 = 128
G1 = 16
HH = 128

NH = N // 2
NHP = NH + 8
NC = 2
NS = 16
SCB = 64
ECH1 = E // SCB
ECPS1 = -(-ECH1 // NS)
E2 = E // NC
ECH2 = E2 // SCB
ECPS2 = -(-ECH2 // NS)
CP = 40
NCH = NH // CP
CPS = -(-NCH // NS)

_DOT = jnp.dot


def _relu(v):
    return jnp.maximum(v, 0.0)


def _proj1_body(x_ref, ws_ref, wr_ref, xs_ref, xr_ref):
    m = _DOT(x_ref[...], ws_ref[...])
    xs_ref[0] = m[:, :HH]
    xs_ref[1] = m[:, HH:]
    m = _DOT(x_ref[...], wr_ref[...])
    xr_ref[0] = m[:, :HH]
    xr_ref[1] = m[:, HH:]


def _zedge_body(ea_ref, we_ref, u_ref, wg_ref, be_ref, z_ref):
    c1 = _DOT(u_ref[...], wg_ref[...]) + be_ref[...]
    z = _DOT(ea_ref[...], we_ref[...]) + c1
    z_ref[0] = z[:, :HH]
    z_ref[1] = z[:, HH:]


def _node1_body(x_ref, seg_ref, cnt_ref, u_ref, wn_ref, win_ref, wgn_ref,
                bn_ref, ws2_ref, wr2_ref,
                n1_ref, ns2_ref, nr2_ref, nsum_ref, esum_ref):
    i = pl.program_id(0)

    @pl.when(i == 0)
    def _():
        nsum_ref[...] = jnp.zeros_like(nsum_ref)
        esum_ref[...] = jnp.zeros_like(esum_ref)

    cb = jnp.maximum(cnt_ref[:, 0:1], 1.0)
    agg_lo = seg_ref[0] / cb
    agg_hi = seg_ref[1] / cb
    pre = (_DOT(x_ref[...], wn_ref[...])
           + _DOT(agg_lo, win_ref[:HH, :])
           + _DOT(agg_hi, win_ref[HH:, :])
           + _DOT(u_ref[...], wgn_ref[...]) + bn_ref[...])
    n1 = _relu(pre)
    n1_ref[...] = n1
    ns2_ref[...] = _DOT(n1, ws2_ref[...])
    nr2_ref[...] = _DOT(n1, wr2_ref[...])
    nsum_ref[...] += jnp.sum(n1, axis=0, keepdims=True)
    esum_ref[...] += jnp.concatenate(
        [jnp.sum(seg_ref[0], axis=0, keepdims=True),
         jnp.sum(seg_ref[1], axis=0, keepdims=True)], axis=1)


def _glob1_body(nsum_ref, esum_ref, u_ref, wgn1_ref, wge1_ref, wgg1_ref,
                bg1_ref, wg2_ref, be2_ref, wgn2_ref, bn2_ref,
                u1_ref, c2_ref, cn2_ref):
    u1 = _relu(_DOT(nsum_ref[...] * (1.0 / N), wgn1_ref[...])
               + _DOT(esum_ref[...] * (1.0 / E), wge1_ref[...])
               + _DOT(u_ref[...], wgg1_ref[...]) + bg1_ref[...])
    u1_ref[...] = u1
    c2_ref[...] = _DOT(u1, wg2_ref[...]) + be2_ref[...]
    cn2_ref[...] = _DOT(u1, wgn2_ref[...]) + bn2_ref[...]


def _t2_body(e1_ref, we2_ref, c2_ref, t2_ref):
    t2_ref[...] = (_DOT(e1_ref[0], we2_ref[:HH, :])
                   + _DOT(e1_ref[1], we2_ref[HH:, :]) + c2_ref[...])


def _node2_body(n1_ref, seg_ref, cnt_ref, wn2_ref, win2_ref, cn2_ref,
                n2sum_ref, esum2_ref):
    i = pl.program_id(0)

    @pl.when(i == 0)
    def _():
        n2sum_ref[...] = jnp.zeros_like(n2sum_ref)
        esum2_ref[...] = jnp.zeros_like(esum2_ref)

    cb = jnp.maximum(cnt_ref[:, 0:1], 1.0)
    sseg = seg_ref[0] + seg_ref[1]
    agg2 = sseg / cb
    n2 = _relu(_DOT(n1_ref[...], wn2_ref[...])
               + _DOT(agg2, win2_ref[...]) + cn2_ref[...])
    n2sum_ref[...] += jnp.sum(n2, axis=0, keepdims=True)
    esum2_ref[...] += jnp.sum(sseg, axis=0, keepdims=True)


def _out_body(n2sum_ref, esum2_ref, u1_ref, wgn2_ref, wge2_ref, wgg2_ref,
              bg2_ref, o_ref):
    o_ref[...] = (_DOT(n2sum_ref[...] * (1.0 / N), wgn2_ref[...])
                  + _DOT(esum2_ref[...] * (1.0 / E), wge2_ref[...])
                  + _DOT(u1_ref[...], wgg2_ref[...]) + bg2_ref[...])


def _full(shape):
    return pl.BlockSpec(shape, lambda *_: (0,) * len(shape))


def _zero_vmem(ref, rows, cols):
    def body(j, carry):
        for k in range(cols // 16):
            ref[j, pl.ds(k * 16, 16)] = jnp.zeros((16,), jnp.float32)
        return carry
    lax.fori_loop(0, rows, body, 0)


def _sc_pass1(z_hbm, xs_hbm, xr_hbm, snd2_hbm, rcv2_hbm, rcvc_hbm,
              e1_hbm, seg_hbm,
              acc_sh,
              sidx, gidx, cidx,
              zb, xsb, xrb,
              sem0, sem1, sem2):
    c = lax.axis_index("c")
    s = lax.axis_index("s")

    def init_acc():
        for kk in range(CPS):
            t = s * CPS + kk

            @pl.when(t < NCH)
            def _():
                pltpu.sync_copy(zb.at[pl.ds(0, CP)],
                                acc_sh.at[pl.ds(t * CP, CP)])

    _zero_vmem(zb, CP, HH)
    init_acc()
    plsc.subcore_barrier()

    for h in range(2):

        def step(i, carry):
            t = s * ECPS1 + i

            @pl.when(t < ECH1)
            def _():
                base = t * SCB
                pltpu.sync_copy(rcvc_hbm.at[pl.ds(h * E + base, SCB)], cidx)
                if h == 0:
                    pltpu.sync_copy(snd2_hbm.at[pl.ds(c * E + base, SCB)],
                                    sidx)
                    pltpu.sync_copy(rcv2_hbm.at[pl.ds(c * E + base, SCB)],
                                    gidx)
                    cp1 = pltpu.async_copy(xs_hbm.at[sidx], xsb, sem0)
                    cp2 = pltpu.async_copy(xr_hbm.at[gidx], xrb, sem1)
                    cp3 = pltpu.async_copy(
                        z_hbm.at[pl.ds(c * E + base, SCB)], zb, sem2)
                    cp1.wait()
                    cp2.wait()
                    cp3.wait()

                    def row(j, carry2):
                        for k in range(HH // 16):
                            sl = pl.ds(k * 16, 16)
                            zb[j, sl] = jnp.maximum(
                                zb[j, sl] + xsb[j, sl] + xrb[j, sl], 0.0)
                        return carry2
                    lax.fori_loop(0, SCB, row, 0)

                    pltpu.sync_copy(zb, e1_hbm.at[pl.ds(c * E + base, SCB)])
                else:
                    pltpu.sync_copy(e1_hbm.at[pl.ds(c * E + base, SCB)], zb)

                pltpu.sync_copy(zb, acc_sh.at[cidx], add=True)
            return carry

        lax.fori_loop(0, ECPS1, step, 0)
        plsc.subcore_barrier()

        for kk in range(CPS):
            t = s * CPS + kk

            @pl.when(t < NCH)
            def _():
                pltpu.sync_copy(
                    acc_sh.at[pl.ds(t * CP, CP)],
                    seg_hbm.at[pl.ds(c * N + h * NH + t * CP, CP)])

        if h == 0:
            _zero_vmem(zb, CP, HH)
            init_acc()
            plsc.subcore_barrier()


def _sc_pass2(t2_hbm, ns2_hbm, nr2_hbm, snd_hbm, rcv_hbm, rcvc_hbm,
              e2_hbm, seg_hbm,
              acc_sh,
              sidx, gidx, cidx,
              tb, nsb, nrb,
              sem0, sem1, sem2):
    c = lax.axis_index("c")
    s = lax.axis_index("s")

    def init_acc():
        for kk in range(CPS):
            t = s * CPS + kk

            @pl.when(t < NCH)
            def _():
                pltpu.sync_copy(tb.at[pl.ds(0, CP)],
                                acc_sh.at[pl.ds(t * CP, CP)])

    _zero_vmem(tb, CP, HH)
    init_acc()
    plsc.subcore_barrier()

    for h in range(2):

        def step(i, carry):
            t = s * ECPS2 + i

            @pl.when(t < ECH2)
            def _():
                base = c * E2 + t * SCB
                pltpu.sync_copy(rcvc_hbm.at[pl.ds(h * E + base, SCB)], cidx)
                if h == 0:
                    pltpu.sync_copy(snd_hbm.at[pl.ds(base, SCB)], sidx)
                    pltpu.sync_copy(rcv_hbm.at[pl.ds(base, SCB)], gidx)
                    cp1 = pltpu.async_copy(ns2_hbm.at[sidx], nsb, sem0)
                    cp2 = pltpu.async_copy(nr2_hbm.at[gidx], nrb, sem1)
                    cp3 = pltpu.async_copy(t2_hbm.at[pl.ds(base, SCB)], tb,
                                           sem2)
                    cp1.wait()
                    cp2.wait()
                    cp3.wait()

                    def row(j, carry2):
                        for k in range(HH // 16):
                            sl = pl.ds(k * 16, 16)
                            tb[j, sl] = jnp.maximum(
                                tb[j, sl] + nsb[j, sl] + nrb[j, sl], 0.0)
                        return carry2
                    lax.fori_loop(0, SCB, row, 0)

                    pltpu.sync_copy(tb, e2_hbm.at[pl.ds(base, SCB)])
                else:
                    pltpu.sync_copy(e2_hbm.at[pl.ds(base, SCB)], tb)

                pltpu.sync_copy(tb, acc_sh.at[cidx], add=True)
            return carry

        lax.fori_loop(0, ECPS2, step, 0)
        plsc.subcore_barrier()

        for kk in range(CPS):
            t = s * CPS + kk

            @pl.when(t < NCH)
            def _():
                pltpu.sync_copy(
                    acc_sh.at[pl.ds(t * CP, CP)],
                    seg_hbm.at[pl.ds(c * N + h * NH + t * CP, CP)])

        if h == 0:
            _zero_vmem(tb, CP, HH)
            init_acc()
            plsc.subcore_barrier()


def _sc_cnt(rcvc_hbm, cnt_hbm, acc_sh, cidx, onesb, sem0):
    c = lax.axis_index("c")
    s = lax.axis_index("s")

    _zero_vmem(onesb, CP, HH)
    for kk in range(CPS):
        t = s * CPS + kk

        @pl.when(t < NCH)
        def _():
            pltpu.sync_copy(onesb.at[pl.ds(0, CP)],
                            acc_sh.at[pl.ds(t * CP, CP)])

    def ones_body(j, carry):
        for k in range(HH // 16):
            onesb[j, pl.ds(k * 16, 16)] = jnp.full((16,), 1.0, jnp.float32)
        return carry
    lax.fori_loop(0, SCB, ones_body, 0)
    plsc.subcore_barrier()

    def step(i, carry):
        t = s * ECPS1 + i

        @pl.when(t < ECH1)
        def _():
            base = t * SCB
            pltpu.sync_copy(rcvc_hbm.at[pl.ds(c * E + base, SCB)], cidx)
            pltpu.sync_copy(onesb, acc_sh.at[cidx], add=True)
        return carry

    lax.fori_loop(0, ECPS1, step, 0)
    plsc.subcore_barrier()

    for kk in range(CPS):
        t = s * CPS + kk

        @pl.when(t < NCH)
        def _():
            pltpu.sync_copy(acc_sh.at[pl.ds(t * CP, CP)],
                            cnt_hbm.at[pl.ds(c * NH + t * CP, CP)])


_sc_calls = {}


def _build_sc_calls():
    if _sc_calls:
        return
    mesh = plsc.VectorSubcoreMesh(core_axis_name="c", subcore_axis_name="s")
    _sc_calls["cnt"] = pl.kernel(
        _sc_cnt,
        out_type=[
            jax.ShapeDtypeStruct((N, HH), jnp.float32),
        ],
        mesh=mesh,
        scratch_types=[
            pltpu.VMEM_SHARED((NHP, HH), jnp.float32),
            pltpu.VMEM((SCB,), jnp.int32),
            pltpu.VMEM((SCB, HH), jnp.float32),
            pltpu.SemaphoreType.DMA,
        ],
    )
    _sc_calls["p1"] = pl.kernel(
        _sc_pass1,
        out_type=[
            jax.ShapeDtypeStruct((2 * E, HH), jnp.float32),
            jax.ShapeDtypeStruct((2 * N, HH), jnp.float32),
        ],
        mesh=mesh,
        scratch_types=[
            pltpu.VMEM_SHARED((NHP, HH), jnp.float32),
            pltpu.VMEM((SCB,), jnp.int32),
            pltpu.VMEM((SCB,), jnp.int32),
            pltpu.VMEM((SCB,), jnp.int32),
            pltpu.VMEM((SCB, HH), jnp.float32),
            pltpu.VMEM((SCB, HH), jnp.float32),
            pltpu.VMEM((SCB, HH), jnp.float32),
            pltpu.SemaphoreType.DMA,
            pltpu.SemaphoreType.DMA,
            pltpu.SemaphoreType.DMA,
        ],
    )
    _sc_calls["p2"] = pl.kernel(
        _sc_pass2,
        out_type=[
            jax.ShapeDtypeStruct((E, HH), jnp.float32),
            jax.ShapeDtypeStruct((2 * N, HH), jnp.float32),
        ],
        mesh=mesh,
        scratch_types=[
            pltpu.VMEM_SHARED((NHP, HH), jnp.float32),
            pltpu.VMEM((SCB,), jnp.int32),
            pltpu.VMEM((SCB,), jnp.int32),
            pltpu.VMEM((SCB,), jnp.int32),
            pltpu.VMEM((SCB, HH), jnp.float32),
            pltpu.VMEM((SCB, HH), jnp.float32),
            pltpu.VMEM((SCB, HH), jnp.float32),
            pltpu.SemaphoreType.DMA,
            pltpu.SemaphoreType.DMA,
            pltpu.SemaphoreType.DMA,
        ],
    )


def kernel(x, edge_attr, senders, receivers, u,
           We1, Ws1, Wr1, Wg1, be1,
           Wn1, Win1, Wgn1, bn1,
           WGn1, WGe1, WGg1, bg1,
           We2, Ws2, Wr2, Wg2, be2,
           Wn2, Win2, Wgn2, bn2,
           WGn2, WGe2, WGg2, bg2):
    _build_sc_calls()
    u2 = u.reshape(1, DG)
    be1r = be1.reshape(1, H1)
    bn1r = bn1.reshape(1, H1)
    bg1r = bg1.reshape(1, G1)
    be2r = be2.reshape(1, H2)
    bn2r = bn2.reshape(1, H2)
    bg2r = bg2.reshape(1, 1)
    snd = senders.astype(jnp.int32)
    rcv = receivers.astype(jnp.int32)
    snd2 = jnp.concatenate([snd, snd + N])
    rcv2 = jnp.concatenate([rcv, rcv + N])
    rcvc = jnp.concatenate([jnp.where(rcv < NH, rcv, NH),
                            jnp.where(rcv >= NH, rcv - NH, NH)])

    BN = 1000
    BEB = 2000

    xs3, xr3 = pl.pallas_call(
        _proj1_body,
        grid=(N // BN,),
        in_specs=[pl.BlockSpec((BN, DN), lambda i: (i, 0)),
                  _full((DN, H1)), _full((DN, H1))],
        out_specs=[pl.BlockSpec((2, BN, HH), lambda i: (0, i, 0)),
                   pl.BlockSpec((2, BN, HH), lambda i: (0, i, 0))],
        out_shape=[jax.ShapeDtypeStruct((2, N, HH), jnp.float32),
                   jax.ShapeDtypeStruct((2, N, HH), jnp.float32)],
    )(x, Ws1, Wr1)

    z3 = pl.pallas_call(
        _zedge_body,
        grid=(E // BEB,),
        in_specs=[pl.BlockSpec((BEB, DE), lambda i: (i, 0)),
                  _full((DE, H1)), _full((1, DG)), _full((DG, H1)),
                  _full((1, H1))],
        out_specs=pl.BlockSpec((2, BEB, HH), lambda i: (0, i, 0)),
        out_shape=jax.ShapeDtypeStruct((2, E, HH), jnp.float32),
    )(edge_attr, We1, u2, Wg1, be1r)

    (cnt,) = _sc_calls["cnt"](rcvc)

    e1f, seg1f = _sc_calls["p1"](
        z3.reshape(2 * E, HH), xs3.reshape(2 * N, HH), xr3.reshape(2 * N, HH),
        snd2, rcv2, rcvc)
    e1_3 = e1f.reshape(2, E, HH)
    seg1_3 = seg1f.reshape(2, N, HH)

    n1, ns2, nr2, nsum, esum1 = pl.pallas_call(
        _node1_body,
        grid=(N // BN,),
        in_specs=[pl.BlockSpec((BN, DN), lambda i: (i, 0)),
                  pl.BlockSpec((2, BN, HH), lambda i: (0, i, 0)),
                  pl.BlockSpec((BN, HH), lambda i: (i, 0)),
                  _full((1, DG)), _full((DN, H1)), _full((H1, H1)),
                  _full((DG, H1)), _full((1, H1)), _full((H1, H2)),
                  _full((H1, H2))],
        out_specs=[pl.BlockSpec((BN, H1), lambda i: (i, 0)),
                   pl.BlockSpec((BN, H2), lambda i: (i, 0)),
                   pl.BlockSpec((BN, H2), lambda i: (i, 0)),
                   pl.BlockSpec((1, H1), lambda i: (0, 0)),
                   pl.BlockSpec((1, H1), lambda i: (0, 0))],
        out_shape=[jax.ShapeDtypeStruct((N, H1), jnp.float32),
                   jax.ShapeDtypeStruct((N, H2), jnp.float32),
                   jax.ShapeDtypeStruct((N, H2), jnp.float32),
                   jax.ShapeDtypeStruct((1, H1), jnp.float32),
                   jax.ShapeDtypeStruct((1, H1), jnp.float32)],
    )(x, seg1_3, cnt, u2, Wn1, Win1, Wgn1, bn1r, Ws2, Wr2)

    u1, c2, cn2 = pl.pallas_call(
        _glob1_body,
        in_specs=[_full((1, H1)), _full((1, H1)), _full((1, DG)),
                  _full((H1, G1)), _full((H1, G1)), _full((DG, G1)),
                  _full((1, G1)), _full((G1, H2)), _full((1, H2)),
                  _full((G1, H2)), _full((1, H2))],
        out_specs=[_full((1, G1)), _full((1, H2)), _full((1, H2))],
        out_shape=[jax.ShapeDtypeStruct((1, G1), jnp.float32),
                   jax.ShapeDtypeStruct((1, H2), jnp.float32),
                   jax.ShapeDtypeStruct((1, H2), jnp.float32)],
    )(nsum, esum1, u2, WGn1, WGe1, WGg1, bg1r, Wg2, be2r, Wgn2, bn2r)

    t2 = pl.pallas_call(
        _t2_body,
        grid=(E // BEB,),
        in_specs=[pl.BlockSpec((2, BEB, HH), lambda i: (0, i, 0)),
                  _full((H1, H2)), _full((1, H2))],
        out_specs=pl.BlockSpec((BEB, H2), lambda i: (i, 0)),
        out_shape=jax.ShapeDtypeStruct((E, H2), jnp.float32),
    )(e1_3, We2, c2)

    _, seg2f = _sc_calls["p2"](t2, ns2, nr2, snd, rcv, rcvc)
    seg2_2 = seg2f.reshape(2, N, HH)

    n2sum, esum2 = pl.pallas_call(
        _node2_body,
        grid=(N // BN,),
        in_specs=[pl.BlockSpec((BN, H1), lambda i: (i, 0)),
                  pl.BlockSpec((2, BN, HH), lambda i: (0, i, 0)),
                  pl.BlockSpec((BN, HH), lambda i: (i, 0)),
                  _full((H1, H2)), _full((H2, H2)), _full((1, H2))],
        out_specs=[pl.BlockSpec((1, H2), lambda i: (0, 0)),
                   pl.BlockSpec((1, H2), lambda i: (0, 0))],
        out_shape=[jax.ShapeDtypeStruct((1, H2), jnp.float32),
                   jax.ShapeDtypeStruct((1, H2), jnp.float32)],
    )(n1, seg2_2, cnt, Wn2, Win2, cn2)

    o = pl.pallas_call(
        _out_body,
        in_specs=[_full((1, H2)), _full((1, H2)), _full((1, G1)),
                  _full((H2, 1)), _full((H2, 1)), _full((G1, 1)),
                  _full((1, 1))],
        out_specs=_full((1, 1)),
        out_shape=jax.ShapeDtypeStruct((1, 1), jnp.float32),
    )(n2sum, esum2, u1, WGn2, WGe2, WGg2, bg2r)

    return o.reshape(1)
